# baseline (speedup 1.0000x reference)
import functools

import jax
import jax.numpy as jnp
from jax import lax
from jax.experimental import pallas as pl
from jax.experimental.pallas import tpu as pltpu
from jax.experimental.pallas import tpu_sc as plsc

_EMB_DIM = 64
_BATCH = 4096
_NUM_TABLES = 26
_OUT_D = (_NUM_TABLES + 1) * _EMB_DIM


def _mlp(dense, W0, b0, W1, b1, W2, b2):
    x = jnp.pad(dense, ((0, 0), (0, 3)))
    W0p = jnp.pad(W0, ((0, 3), (0, 0)))

    def body(x_ref, w0_ref, b0_ref, w1_ref, b1_ref, w2_ref, b2_ref, o_ref):
        h = jnp.dot(x_ref[...], w0_ref[...], preferred_element_type=jnp.float32)
        h = jnp.maximum(h + b0_ref[...], 0.0)
        h = jnp.dot(h, w1_ref[...], preferred_element_type=jnp.float32)
        h = jnp.maximum(h + b1_ref[...], 0.0)
        h = jnp.dot(h, w2_ref[...], preferred_element_type=jnp.float32)
        o_ref[...] = h + b2_ref[...]

    TB = 512
    return pl.pallas_call(
        body,
        grid=(_BATCH // TB,),
        in_specs=[
            pl.BlockSpec((TB, 16), lambda i: (i, 0)),
            pl.BlockSpec((16, 512), lambda i: (0, 0)),
            pl.BlockSpec((1, 512), lambda i: (0, 0)),
            pl.BlockSpec((512, 256), lambda i: (0, 0)),
            pl.BlockSpec((1, 256), lambda i: (0, 0)),
            pl.BlockSpec((256, 64), lambda i: (0, 0)),
            pl.BlockSpec((1, 64), lambda i: (0, 0)),
        ],
        out_specs=pl.BlockSpec((TB, _EMB_DIM), lambda i: (i, 0)),
        out_shape=jax.ShapeDtypeStruct((_BATCH, _EMB_DIM), jnp.float32),
    )(x, W0p, b0.reshape(1, -1), W1, b1.reshape(1, -1), W2, b2.reshape(1, -1))


def _sc_gather_concat(cat_all, tables, mlp_out):
    info = plsc.get_sparse_core_info()
    nc, ns = info.num_cores, info.num_subcores
    nw = nc * ns
    bpw = _BATCH // nw

    mesh = plsc.VectorSubcoreMesh(core_axis_name="c", subcore_axis_name="s")

    @functools.partial(
        pl.kernel,
        out_type=jax.ShapeDtypeStruct((_BATCH, _OUT_D), jnp.float32),
        mesh=mesh,
        scratch_types=[
            pltpu.VMEM((bpw,), jnp.int32),
            pltpu.VMEM((bpw, _EMB_DIM), jnp.float32),
            pltpu.SemaphoreType.DMA,
        ],
        compiler_params=pltpu.CompilerParams(use_tc_tiling_on_sc=False),
    )
    def k(cat_ref, *rest):
        tab_refs = rest[:_NUM_TABLES]
        mlp_ref = rest[_NUM_TABLES]
        out_ref = rest[_NUM_TABLES + 1]
        idx_v, rows_v, sem = rest[_NUM_TABLES + 2:]

        wid = lax.axis_index("s") * nc + lax.axis_index("c")
        base = wid * bpw
        rows = pl.ds(base, bpw)
        for t in range(_NUM_TABLES):
            pltpu.sync_copy(cat_ref.at[t, rows], idx_v)
            pltpu.async_copy(tab_refs[t].at[idx_v], rows_v, sem).wait()
            pltpu.sync_copy(rows_v, out_ref.at[rows, pl.ds(t * _EMB_DIM, _EMB_DIM)])
        pltpu.sync_copy(mlp_ref.at[rows], rows_v)
        pltpu.sync_copy(
            rows_v, out_ref.at[rows, pl.ds(_NUM_TABLES * _EMB_DIM, _EMB_DIM)]
        )

    return k(cat_all, *tables, mlp_out)


def kernel(cat_0, cat_1, cat_2, cat_3, cat_4, cat_5, cat_6, cat_7, cat_8,
           cat_9, cat_10, cat_11, cat_12, cat_13, cat_14, cat_15, cat_16,
           cat_17, cat_18, cat_19, cat_20, cat_21, cat_22, cat_23, cat_24,
           cat_25, dense, emb_0, emb_1, emb_2, emb_3, emb_4, emb_5, emb_6,
           emb_7, emb_8, emb_9, emb_10, emb_11, emb_12, emb_13, emb_14,
           emb_15, emb_16, emb_17, emb_18, emb_19, emb_20, emb_21, emb_22,
           emb_23, emb_24, emb_25, W0, b0, W1, b1, W2, b2):
    kw = dict(locals())
    cats = [kw['cat_%d' % i].astype(jnp.int32) for i in range(_NUM_TABLES)]
    tables = [kw['emb_%d' % i] for i in range(_NUM_TABLES)]
    cat_all = jnp.stack(cats, axis=0)
    mlp_out = _mlp(dense, W0, b0, W1, b1, W2, b2)
    return _sc_gather_concat(cat_all, tables, mlp_out)
